# baseline (speedup 1.0000x reference)
import functools

import jax
import jax.numpy as jnp
from jax import lax
from jax.experimental import pallas as pl
from jax.experimental.pallas import tpu as pltpu
from jax.experimental.pallas import tpu_sc as plsc

MAX_DEG = 192
B = 192
NEG = -1e30
CH = 128
NW = 32


def _proj_body(x_ref, wl_ref, bl_ref, w0_ref, w1_ref, w2_ref, eb_ref, qkv_ref):
    x = x_ref[...]
    d = x.shape[1]
    h = jnp.dot(x, wl_ref[...], preferred_element_type=jnp.float32) + bl_ref[...]
    eb = eb_ref[...]
    qkv_ref[:, 0:d] = jnp.dot(h, w0_ref[...], preferred_element_type=jnp.float32) + eb[0:1, :]
    qkv_ref[:, d:2 * d] = jnp.dot(h, w1_ref[...], preferred_element_type=jnp.float32) + eb[1:2, :]
    qkv_ref[:, 2 * d:3 * d] = jnp.dot(h, w2_ref[...], preferred_element_type=jnp.float32) + eb[2:3, :]


def _gather_body(table_hbm, idx_hbm, out_hbm, idx_v, rows_v, sem):
    wid = lax.axis_index("s") * 2 + lax.axis_index("c")
    nchunks = idx_hbm.shape[0] // CH
    nmine = (nchunks - wid + NW - 1) // NW

    def body(i, _):
        base = (wid + i * NW) * CH
        pltpu.sync_copy(idx_hbm.at[pl.ds(base, CH)], idx_v)
        pltpu.async_copy(table_hbm.at[idx_v], rows_v, sem).wait()
        pltpu.sync_copy(rows_v, out_hbm.at[pl.ds(base, CH)])
        return 0

    lax.fori_loop(0, nmine, body, 0)


def _segreduce_body(split, np2, nacc, s_hbm, dst_hbm, pos_hbm, vps_hbm,
                    mvec_hbm, cb_hbm, out_hbm, idx_v, iw_v, tl_v, s_v, pos_v,
                    w_v, rows_v, scaled_v, wrow_v, mvec_v, cb_v, acc_sh):
    cid = lax.axis_index("c")
    sid = lax.axis_index("s")
    nchunks = s_hbm.shape[0] // CH
    stripe = nacc // 16
    tb = np2 + np2 // 8
    r0 = sid * stripe

    def zrow(r, _):
        for j in range(8):
            scaled_v[r, pl.ds(j * 16, 16)] = jnp.zeros((16,), jnp.float32)
        return 0
    lax.fori_loop(0, CH, zrow, 0)
    for kk in range(stripe // CH):
        pltpu.sync_copy(scaled_v, acc_sh.at[pl.ds(r0 + kk * CH, CH)])
    plsc.subcore_barrier()

    pltpu.sync_copy(mvec_hbm, mvec_v)
    pltpu.sync_copy(cb_hbm, cb_v)
    cb = cb_v[pl.ds(0, 1)][0]
    start = cid * jnp.maximum(cb - 1, 0)
    end = cb + cid * (nchunks - cb)
    nmine = (end - start - sid + 15) // 16
    base_v = lax.broadcast_in_dim(cid * split, (16,), ())
    lane = lax.iota(jnp.int32, 16)
    zero16 = jnp.zeros((16,), jnp.float32)

    def chunk(i, _):
        base = (start + sid + i * 16) * CH
        pltpu.sync_copy(dst_hbm.at[pl.ds(base, CH)], idx_v)
        pltpu.sync_copy(s_hbm.at[pl.ds(base, CH)], s_v)
        pltpu.sync_copy(pos_hbm.at[pl.ds(base, CH)], pos_v)
        pltpu.sync_copy(vps_hbm.at[pl.ds(base, CH)], rows_v)
        mv = mvec_v[...]
        for g in range(CH // 16):
            sg = s_v[pl.ds(g * 16, 16)]
            pg = pos_v[pl.ds(g * 16, 16)]
            dg = idx_v[pl.ds(g * 16, 16)]
            w_v[pl.ds(g * 16, 16)] = jnp.where(pg < MAX_DEG, jnp.exp(sg - mv), 0.0)
            rel = dg - base_v
            ok = (rel >= 0) & (rel < np2)
            trash = tb + (dg & 127)
            idx_v[pl.ds(g * 16, 16)] = jnp.where(ok, rel, trash)
            iw_v[pl.ds(g * 16, 16)] = jnp.where(
                ok, np2 + lax.shift_right_logical(rel, 3), trash)
            tl_v[pl.ds(g * 16, 16)] = (rel & 7) * 16

        def edge(ei, _):
            ws = w_v[pl.ds(ei, 1)][0]
            tl = tl_v[pl.ds(ei, 1)][0]
            wsb = lax.broadcast_in_dim(ws, (16,), ())
            tlb = lax.broadcast_in_dim(tl, (16,), ())
            for j in range(8):
                scaled_v[ei, pl.ds(j * 16, 16)] = rows_v[ei, pl.ds(j * 16, 16)] * wsb
                wrow_v[ei, pl.ds(j * 16, 16)] = jnp.where(
                    lane == tlb - 16 * j, wsb, zero16)
            return 0
        lax.fori_loop(0, CH, edge, 0)
        pltpu.sync_copy(scaled_v, acc_sh.at[idx_v], add=True)
        pltpu.sync_copy(wrow_v, acc_sh.at[iw_v], add=True)
        return 0

    lax.fori_loop(0, nmine, chunk, 0)
    plsc.subcore_barrier()
    for kk in range(stripe // CH):
        pltpu.sync_copy(acc_sh.at[pl.ds(r0 + kk * CH, CH)], scaled_v)
        pltpu.sync_copy(scaled_v, out_hbm.at[cid, pl.ds(r0 + kk * CH, CH)])


def _attn_body(nb, g_ref, gp_ref, gn_ref, mq_ref, mkp_ref, mkc_ref, mkn_ref,
               w3_ref, wl_ref, w2_ref, bp_ref, vout_ref, sout_ref, mout_ref,
               msc_ref):
    b = pl.program_id(0)
    d = w3_ref.shape[0]
    g = g_ref[...]
    q = g[:, 0:d]
    k = jnp.concatenate([gp_ref[:, d:2 * d], g[:, d:2 * d], gn_ref[:, d:2 * d]], axis=0)
    v = jnp.concatenate([gp_ref[:, 2 * d:], g[:, 2 * d:], gn_ref[:, 2 * d:]], axis=0)
    mq = mq_ref[0]
    dq = mq[:, 0:1]
    pq = mq[:, 1:2]
    mk = jnp.concatenate([mkp_ref[0], mkc_ref[0], mkn_ref[0]], axis=1)
    dk = mk[0:1, :]
    pk = mk[1:2, :]
    cols = lax.broadcasted_iota(jnp.int32, (1, 3 * B), 1)
    okl = (b > 0).astype(jnp.int32)
    okr = (b < nb - 1).astype(jnp.int32)
    band = jnp.where(cols < B, okl, jnp.where(cols >= 2 * B, okr, 1))
    mask = (dq == dk) & (pk < MAX_DEG) & (band > 0)
    scores = lax.dot_general(q.astype(jnp.bfloat16), k.astype(jnp.bfloat16),
                             (((1,), (1,)), ((), ())),
                             preferred_element_type=jnp.float32)
    scores = scores * (1.0 / jnp.sqrt(jnp.float32(d)))
    scores = jnp.where(mask, scores, NEG)
    m = jnp.max(scores, axis=1, keepdims=True)
    ex = jnp.exp(scores - m)
    den = jnp.sum(ex, axis=1, keepdims=True)
    att = jnp.dot(ex.astype(jnp.bfloat16), v.astype(jnp.bfloat16),
                  preferred_element_type=jnp.float32) / den
    out = q + att
    bp = bp_ref[...]
    xe = out + jax.nn.relu(
        jnp.dot(out, w3_ref[...], preferred_element_type=jnp.float32) + bp[0:1, :])
    kp = jax.nn.relu(
        jnp.dot(xe, wl_ref[...], preferred_element_type=jnp.float32) + bp[1:2, :])
    vout_ref[...] = jnp.dot(kp, w2_ref[...], preferred_element_type=jnp.float32) + bp[2:3, :]
    s = jnp.sum(kp * bp[3:4, :], axis=1, keepdims=True) + bp[4:5, 0:1]
    sout_ref[...] = s
    blk_m = jnp.max(jnp.where(pq < MAX_DEG, s, NEG))

    @pl.when(b == 0)
    def _():
        msc_ref[0, 0] = NEG
    msc_ref[0, 0] = jnp.maximum(msc_ref[0, 0], blk_m)

    @pl.when(b == nb - 1)
    def _():
        mout_ref[...] = jnp.broadcast_to(msc_ref[0, 0], (1, mout_ref.shape[1]))


def _tail_body(acc_ref, den_ref, qs_ref, w3_ref, wd_ref, wd3_ref, bp_ref, o_ref):
    bp = bp_ref[...]
    a = acc_ref[0]
    den = den_ref[:, 0:1]
    pooled = jnp.where(den > 0.0, a / jnp.maximum(den, 1e-30), bp[3:4, :])
    o = qs_ref[...] + pooled
    xp = o + jax.nn.relu(
        jnp.dot(o, w3_ref[...], preferred_element_type=jnp.float32) + bp[0:1, :])
    t = jnp.dot(xp, wd_ref[...], preferred_element_type=jnp.float32) + bp[1:2, :]
    xd = t + jax.nn.relu(
        jnp.dot(t, wd3_ref[...], preferred_element_type=jnp.float32) + bp[2:3, :])
    xd = jnp.where(jnp.isnan(xd), 0.0, xd)
    o_ref[...] = jnp.clip(xd, 0.0, 3.4028235e38)


def kernel(x, edge_index, W_lin, b_lin, enc_W, enc_b, pma_lin_W, pma_lin_b,
           pma_S, pma_W, pma_b, dec_W, dec_b):
    n, d = x.shape
    e = edge_index.shape[1]
    ep = -(-e // 384) * 384
    nb = ep // B

    src, dst = edge_index[0], edge_index[1]
    order = jnp.argsort(dst)
    dst_s = dst[order]
    src_s = src[order]
    counts = jnp.bincount(dst, length=n)
    starts = jnp.cumsum(counts) - counts
    pos = jnp.arange(e, dtype=jnp.int32) - starts[dst_s].astype(jnp.int32)
    pad = ep - e
    dst_p = jnp.concatenate([dst_s.astype(jnp.int32),
                             jnp.full((pad,), n, jnp.int32)])
    src_p = jnp.concatenate([src_s.astype(jnp.int32),
                             jnp.zeros((pad,), jnp.int32)])
    pos_p = jnp.concatenate([pos, jnp.full((pad,), MAX_DEG, jnp.int32)])

    qseed = (pma_S[0] @ pma_W[0] + pma_b[0])
    inv = 1.0 / jnp.sqrt(jnp.float32(d))
    u = (pma_W[1] @ qseed[0]) * inv
    c = (pma_b[1] @ qseed[0]) * inv
    bias_pack = jnp.zeros((8, d), jnp.float32)
    bias_pack = bias_pack.at[0].set(enc_b[3]).at[1].set(pma_lin_b)
    bias_pack = bias_pack.at[2].set(pma_b[2]).at[3].set(u).at[4, 0].set(c)
    tail_bias = jnp.zeros((8, d), jnp.float32)
    tail_bias = tail_bias.at[0].set(pma_b[3]).at[1].set(dec_b[0] + dec_b[2])
    tail_bias = tail_bias.at[2].set(dec_b[3])
    o0 = enc_b[0] + enc_b[2]
    xe0 = o0 + jax.nn.relu(o0 @ enc_W[3] + enc_b[3])
    kp0 = jax.nn.relu(xe0 @ pma_lin_W + pma_lin_b)
    tail_bias = tail_bias.at[3].set(kp0 @ pma_W[2] + pma_b[2])
    wd02 = dec_W[0] + dec_W[2]

    rb = 2000 if n % 2000 == 0 else n
    qkv = pl.pallas_call(
        _proj_body,
        grid=(n // rb,),
        in_specs=[
            pl.BlockSpec((rb, d), lambda i: (i, 0)),
            pl.BlockSpec((d, d), lambda i: (0, 0)),
            pl.BlockSpec((1, d), lambda i: (0, 0)),
            pl.BlockSpec((d, d), lambda i: (0, 0)),
            pl.BlockSpec((d, d), lambda i: (0, 0)),
            pl.BlockSpec((d, d), lambda i: (0, 0)),
            pl.BlockSpec((4, d), lambda i: (0, 0)),
        ],
        out_specs=pl.BlockSpec((rb, 3 * d), lambda i: (i, 0)),
        out_shape=jax.ShapeDtypeStruct((n, 3 * d), jnp.float32),
    )(x, W_lin, b_lin.reshape(1, d), enc_W[0], enc_W[1], enc_W[2], enc_b)

    mesh = plsc.VectorSubcoreMesh(core_axis_name="c", subcore_axis_name="s")
    g_edge = pl.kernel(
        _gather_body,
        mesh=mesh,
        out_type=jax.ShapeDtypeStruct((ep, 3 * d), jnp.float32),
        scratch_types=[
            pltpu.VMEM((CH,), jnp.int32),
            pltpu.VMEM((CH, 3 * d), jnp.float32),
            pltpu.SemaphoreType.DMA,
        ],
    )(qkv, src_p)

    meta_q = jnp.stack([dst_p.reshape(nb, B), pos_p.reshape(nb, B)], axis=2)
    meta_k = jnp.stack([dst_p.reshape(nb, B), pos_p.reshape(nb, B)], axis=1)

    prev = lambda b: (jnp.maximum(b - 1, 0), 0)
    cur = lambda b: (b, 0)
    nxt = lambda b: (jnp.minimum(b + 1, nb - 1), 0)
    vps, sps, mout = pl.pallas_call(
        functools.partial(_attn_body, nb),
        grid=(nb,),
        in_specs=[
            pl.BlockSpec((B, 3 * d), cur),
            pl.BlockSpec((B, 3 * d), prev),
            pl.BlockSpec((B, 3 * d), nxt),
            pl.BlockSpec((1, B, 2), lambda b: (b, 0, 0)),
            pl.BlockSpec((1, 2, B), lambda b: (jnp.maximum(b - 1, 0), 0, 0)),
            pl.BlockSpec((1, 2, B), lambda b: (b, 0, 0)),
            pl.BlockSpec((1, 2, B), lambda b: (jnp.minimum(b + 1, nb - 1), 0, 0)),
            pl.BlockSpec((d, d), lambda b: (0, 0)),
            pl.BlockSpec((d, d), lambda b: (0, 0)),
            pl.BlockSpec((d, d), lambda b: (0, 0)),
            pl.BlockSpec((8, d), lambda b: (0, 0)),
        ],
        out_specs=[pl.BlockSpec((B, d), cur),
                   pl.BlockSpec((B, 1), lambda b: (b, 0)),
                   pl.BlockSpec((1, d), lambda b: (0, 0))],
        out_shape=[jax.ShapeDtypeStruct((ep, d), jnp.float32),
                   jax.ShapeDtypeStruct((ep, 1), jnp.float32),
                   jax.ShapeDtypeStruct((1, d), jnp.float32)],
        scratch_shapes=[pltpu.SMEM((1, 1), jnp.float32)],
    )(g_edge, g_edge, g_edge, meta_q, meta_k, meta_k, meta_k,
      enc_W[3], pma_lin_W, pma_W[2], bias_pack)

    mvec = mout[0, :16]
    split = n // 2
    np2 = -(-max(split, n + 1 - split) // 128) * 128
    nacc = -(-(np2 + np2 // 8 + 128) // 2048) * 2048
    first_hi = jnp.searchsorted(dst_p, split).astype(jnp.int32)
    cb_vec = jnp.broadcast_to((first_hi + CH - 1) // CH, (16,))
    acc2 = pl.kernel(
        functools.partial(_segreduce_body, split, np2, nacc),
        mesh=mesh,
        out_type=jax.ShapeDtypeStruct((2, nacc, d), jnp.float32),
        scratch_types=[
            pltpu.VMEM((CH,), jnp.int32),
            pltpu.VMEM((CH,), jnp.int32),
            pltpu.VMEM((CH,), jnp.int32),
            pltpu.VMEM((CH,), jnp.float32),
            pltpu.VMEM((CH,), jnp.int32),
            pltpu.VMEM((CH,), jnp.float32),
            pltpu.VMEM((CH, d), jnp.float32),
            pltpu.VMEM((CH, d), jnp.float32),
            pltpu.VMEM((CH, d), jnp.float32),
            pltpu.VMEM((16,), jnp.float32),
            pltpu.VMEM((16,), jnp.int32),
            pltpu.VMEM_SHARED((nacc, d), jnp.float32),
        ],
    )(sps.reshape(ep), dst_p, pos_p, vps, mvec, cb_vec)

    den = acc2[:, np2:np2 + np2 // 8].reshape(2 * np2, 16)

    rb4 = 1024 if np2 % 1024 == 0 else 128
    nb2 = np2 // rb4
    out = pl.pallas_call(
        _tail_body,
        grid=(2 * nb2,),
        in_specs=[
            pl.BlockSpec((1, rb4, d), lambda i: (i // nb2, i % nb2, 0)),
            pl.BlockSpec((rb4, 16), lambda i: (i, 0)),
            pl.BlockSpec((1, d), lambda i: (0, 0)),
            pl.BlockSpec((d, d), lambda i: (0, 0)),
            pl.BlockSpec((d, d), lambda i: (0, 0)),
            pl.BlockSpec((d, d), lambda i: (0, 0)),
            pl.BlockSpec((8, d), lambda i: (0, 0)),
        ],
        out_specs=pl.BlockSpec((rb4, d), lambda i: (i, 0)),
        out_shape=jax.ShapeDtypeStruct((2 * np2, d), jnp.float32),
    )(acc2, den, qseed, pma_W[3], wd02, dec_W[3], tail_bias)
    return jnp.concatenate([out[:split], out[np2:np2 + n - split]])


# speedup vs baseline: 1.1513x; 1.1513x over previous
import functools

import jax
import jax.numpy as jnp
from jax import lax
from jax.experimental import pallas as pl
from jax.experimental.pallas import tpu as pltpu
from jax.experimental.pallas import tpu_sc as plsc

MAX_DEG = 192
B = 192
NEG = -1e30
CH = 128
NW = 32


def _proj_body(x_ref, wl_ref, bl_ref, w0_ref, w1_ref, w2_ref, eb_ref, qkv_ref):
    x = x_ref[...]
    d = x.shape[1]
    h = jnp.dot(x, wl_ref[...], preferred_element_type=jnp.float32) + bl_ref[...]
    eb = eb_ref[...]
    qkv_ref[:, 0:d] = jnp.dot(h, w0_ref[...], preferred_element_type=jnp.float32) + eb[0:1, :]
    qkv_ref[:, d:2 * d] = jnp.dot(h, w1_ref[...], preferred_element_type=jnp.float32) + eb[1:2, :]
    qkv_ref[:, 2 * d:3 * d] = jnp.dot(h, w2_ref[...], preferred_element_type=jnp.float32) + eb[2:3, :]


def _gather_body(table_hbm, idx_hbm, out_hbm, idx_v, rows_v, sem):
    wid = lax.axis_index("s") * 2 + lax.axis_index("c")
    nchunks = idx_hbm.shape[0] // CH
    nmine = (nchunks - wid + NW - 1) // NW

    def body(i, _):
        base = (wid + i * NW) * CH
        pltpu.sync_copy(idx_hbm.at[pl.ds(base, CH)], idx_v)
        pltpu.async_copy(table_hbm.at[idx_v], rows_v, sem).wait()
        pltpu.sync_copy(rows_v, out_hbm.at[pl.ds(base, CH)])
        return 0

    lax.fori_loop(0, nmine, body, 0)


def _segreduce_body(split, np2, nacc, s_hbm, dst_hbm, pos_hbm, vps_hbm,
                    mvec_hbm, cb_hbm, out_hbm, idx_v, iw_v, tl_v, s_v, pos_v,
                    w_v, rows_v, scaled_v, wrow_v, mvec_v, cb_v, acc_sh):
    cid = lax.axis_index("c")
    sid = lax.axis_index("s")
    nchunks = s_hbm.shape[0] // CH
    stripe = nacc // 16
    tb = np2 + np2 // 8
    r0 = sid * stripe

    def zrow(r, _):
        for j in range(8):
            scaled_v[r, pl.ds(j * 16, 16)] = jnp.zeros((16,), jnp.float32)
        return 0
    lax.fori_loop(0, CH, zrow, 0)
    for kk in range(stripe // CH):
        pltpu.sync_copy(scaled_v, acc_sh.at[pl.ds(r0 + kk * CH, CH)])
    plsc.subcore_barrier()

    pltpu.sync_copy(mvec_hbm, mvec_v)
    pltpu.sync_copy(cb_hbm, cb_v)
    cb = cb_v[pl.ds(0, 1)][0]
    start = cid * jnp.maximum(cb - 1, 0)
    end = cb + cid * (nchunks - cb)
    nmine = (end - start - sid + 15) // 16
    base_v = lax.broadcast_in_dim(cid * split, (16,), ())
    lane = lax.iota(jnp.int32, 16)
    zero16 = jnp.zeros((16,), jnp.float32)

    def chunk(i, _):
        base = (start + sid + i * 16) * CH
        pltpu.sync_copy(dst_hbm.at[pl.ds(base, CH)], idx_v)
        pltpu.sync_copy(s_hbm.at[pl.ds(base, CH)], s_v)
        pltpu.sync_copy(pos_hbm.at[pl.ds(base, CH)], pos_v)
        pltpu.sync_copy(vps_hbm.at[pl.ds(base, CH)], rows_v)
        mv = mvec_v[...]
        for g in range(CH // 16):
            sg = s_v[pl.ds(g * 16, 16)]
            pg = pos_v[pl.ds(g * 16, 16)]
            dg = idx_v[pl.ds(g * 16, 16)]
            w_v[pl.ds(g * 16, 16)] = jnp.where(pg < MAX_DEG, jnp.exp(sg - mv), 0.0)
            rel = dg - base_v
            ok = (rel >= 0) & (rel < np2)
            trash = tb + (dg & 127)
            idx_v[pl.ds(g * 16, 16)] = jnp.where(ok, rel, trash)
            iw_v[pl.ds(g * 16, 16)] = jnp.where(
                ok, np2 + lax.shift_right_logical(rel, 3), trash)
            tl_v[pl.ds(g * 16, 16)] = (rel & 7) * 16

        def edge(ei, _):
            ws = w_v[pl.ds(ei, 1)][0]
            tl = tl_v[pl.ds(ei, 1)][0]
            wsb = lax.broadcast_in_dim(ws, (16,), ())
            tlb = lax.broadcast_in_dim(tl, (16,), ())
            for j in range(8):
                scaled_v[ei, pl.ds(j * 16, 16)] = rows_v[ei, pl.ds(j * 16, 16)] * wsb
                wrow_v[ei, pl.ds(j * 16, 16)] = jnp.where(
                    lane == tlb - 16 * j, wsb, zero16)
            return 0
        lax.fori_loop(0, CH, edge, 0)
        pltpu.sync_copy(scaled_v, acc_sh.at[idx_v], add=True)
        pltpu.sync_copy(wrow_v, acc_sh.at[iw_v], add=True)
        return 0

    lax.fori_loop(0, nmine, chunk, 0)
    plsc.subcore_barrier()
    for kk in range(stripe // CH):
        pltpu.sync_copy(acc_sh.at[pl.ds(r0 + kk * CH, CH)], scaled_v)
        pltpu.sync_copy(scaled_v, out_hbm.at[cid, pl.ds(r0 + kk * CH, CH)])


def _attn_body(nb2, gm1_ref, g0_ref, g1_ref, g2_ref, mq0_ref, mq1_ref,
               mkm1_ref, mk0_ref, mk1_ref, mk2_ref,
               w3_ref, wl_ref, w2_ref, bp_ref, vout_ref, sout_ref, mout_ref,
               msc_ref):
    b = pl.program_id(0)
    d = w3_ref.shape[0]
    g0 = g0_ref[...]
    g1 = g1_ref[...]
    q = jnp.concatenate([g0[:, 0:d], g1[:, 0:d]], axis=0)
    k = jnp.concatenate([gm1_ref[:, d:2 * d], g0[:, d:2 * d],
                         g1[:, d:2 * d], g2_ref[:, d:2 * d]], axis=0)
    v = jnp.concatenate([gm1_ref[:, 2 * d:], g0[:, 2 * d:],
                         g1[:, 2 * d:], g2_ref[:, 2 * d:]], axis=0)
    mq = jnp.concatenate([mq0_ref[0], mq1_ref[0]], axis=0)
    dq = mq[:, 0:1]
    pq = mq[:, 1:2]
    mk = jnp.concatenate([mkm1_ref[0], mk0_ref[0], mk1_ref[0], mk2_ref[0]],
                         axis=1)
    dk = mk[0:1, :]
    pk = mk[1:2, :]
    cols = lax.broadcasted_iota(jnp.int32, (1, 4 * B), 1)
    okl = (b > 0).astype(jnp.int32)
    okr = (b < nb2 - 1).astype(jnp.int32)
    band = jnp.where(cols < B, okl, jnp.where(cols >= 3 * B, okr, 1))
    mask = (dq == dk) & (pk < MAX_DEG) & (band > 0)
    scores = lax.dot_general(q, k, (((1,), (1,)), ((), ())),
                             preferred_element_type=jnp.float32)
    scores = scores * (1.0 / jnp.sqrt(jnp.float32(d)))
    scores = jnp.where(mask, scores, NEG)
    m = jnp.max(scores, axis=1, keepdims=True)
    ex = jnp.exp(scores - m)
    den = jnp.sum(ex, axis=1, keepdims=True)
    att = jnp.dot(ex, v, preferred_element_type=jnp.float32) / den
    out = q + att
    bp = bp_ref[...]
    xe = out + jax.nn.relu(
        jnp.dot(out, w3_ref[...], preferred_element_type=jnp.float32) + bp[0:1, :])
    kp = jax.nn.relu(
        jnp.dot(xe, wl_ref[...], preferred_element_type=jnp.float32) + bp[1:2, :])
    vout_ref[...] = jnp.dot(kp, w2_ref[...], preferred_element_type=jnp.float32) + bp[2:3, :]
    s = jnp.sum(kp * bp[3:4, :], axis=1, keepdims=True) + bp[4:5, 0:1]
    sout_ref[...] = s
    blk_m = jnp.max(jnp.where(pq < MAX_DEG, s, NEG))

    @pl.when(b == 0)
    def _():
        msc_ref[0, 0] = NEG
    msc_ref[0, 0] = jnp.maximum(msc_ref[0, 0], blk_m)

    @pl.when(b == nb2 - 1)
    def _():
        mout_ref[...] = jnp.broadcast_to(msc_ref[0, 0], (1, mout_ref.shape[1]))


def _tail_body(acc_ref, den_ref, qs_ref, w3_ref, wd_ref, wd3_ref, bp_ref, o_ref):
    bp = bp_ref[...]
    a = acc_ref[0]
    den = den_ref[:, 0:1]
    pooled = jnp.where(den > 0.0, a / jnp.maximum(den, 1e-30), bp[3:4, :])
    o = qs_ref[...] + pooled
    xp = o + jax.nn.relu(
        jnp.dot(o, w3_ref[...], preferred_element_type=jnp.float32) + bp[0:1, :])
    t = jnp.dot(xp, wd_ref[...], preferred_element_type=jnp.float32) + bp[1:2, :]
    xd = t + jax.nn.relu(
        jnp.dot(t, wd3_ref[...], preferred_element_type=jnp.float32) + bp[2:3, :])
    xd = jnp.where(jnp.isnan(xd), 0.0, xd)
    o_ref[...] = jnp.clip(xd, 0.0, 3.4028235e38)


def kernel(x, edge_index, W_lin, b_lin, enc_W, enc_b, pma_lin_W, pma_lin_b,
           pma_S, pma_W, pma_b, dec_W, dec_b):
    n, d = x.shape
    e = edge_index.shape[1]
    ep = -(-e // 384) * 384
    nb = ep // B

    src, dst = edge_index[0], edge_index[1]
    order = jnp.argsort(dst)
    dst_s = dst[order]
    src_s = src[order]
    counts = jnp.bincount(dst, length=n)
    starts = jnp.cumsum(counts) - counts
    pos = jnp.arange(e, dtype=jnp.int32) - starts[dst_s].astype(jnp.int32)
    pad = ep - e
    dst_p = jnp.concatenate([dst_s.astype(jnp.int32),
                             jnp.full((pad,), n, jnp.int32)])
    src_p = jnp.concatenate([src_s.astype(jnp.int32),
                             jnp.zeros((pad,), jnp.int32)])
    pos_p = jnp.concatenate([pos, jnp.full((pad,), MAX_DEG, jnp.int32)])

    qseed = (pma_S[0] @ pma_W[0] + pma_b[0])
    inv = 1.0 / jnp.sqrt(jnp.float32(d))
    u = (pma_W[1] @ qseed[0]) * inv
    c = (pma_b[1] @ qseed[0]) * inv
    bias_pack = jnp.zeros((8, d), jnp.float32)
    bias_pack = bias_pack.at[0].set(enc_b[3]).at[1].set(pma_lin_b)
    bias_pack = bias_pack.at[2].set(pma_b[2]).at[3].set(u).at[4, 0].set(c)
    tail_bias = jnp.zeros((8, d), jnp.float32)
    tail_bias = tail_bias.at[0].set(pma_b[3]).at[1].set(dec_b[0] + dec_b[2])
    tail_bias = tail_bias.at[2].set(dec_b[3])
    o0 = enc_b[0] + enc_b[2]
    xe0 = o0 + jax.nn.relu(o0 @ enc_W[3] + enc_b[3])
    kp0 = jax.nn.relu(xe0 @ pma_lin_W + pma_lin_b)
    tail_bias = tail_bias.at[3].set(kp0 @ pma_W[2] + pma_b[2])
    wd02 = dec_W[0] + dec_W[2]

    rb = 2000 if n % 2000 == 0 else n
    qkv = pl.pallas_call(
        _proj_body,
        grid=(n // rb,),
        in_specs=[
            pl.BlockSpec((rb, d), lambda i: (i, 0)),
            pl.BlockSpec((d, d), lambda i: (0, 0)),
            pl.BlockSpec((1, d), lambda i: (0, 0)),
            pl.BlockSpec((d, d), lambda i: (0, 0)),
            pl.BlockSpec((d, d), lambda i: (0, 0)),
            pl.BlockSpec((d, d), lambda i: (0, 0)),
            pl.BlockSpec((4, d), lambda i: (0, 0)),
        ],
        out_specs=pl.BlockSpec((rb, 3 * d), lambda i: (i, 0)),
        out_shape=jax.ShapeDtypeStruct((n, 3 * d), jnp.float32),
    )(x, W_lin, b_lin.reshape(1, d), enc_W[0], enc_W[1], enc_W[2], enc_b)

    mesh = plsc.VectorSubcoreMesh(core_axis_name="c", subcore_axis_name="s")
    g_edge = pl.kernel(
        _gather_body,
        mesh=mesh,
        out_type=jax.ShapeDtypeStruct((ep, 3 * d), jnp.float32),
        scratch_types=[
            pltpu.VMEM((CH,), jnp.int32),
            pltpu.VMEM((CH, 3 * d), jnp.float32),
            pltpu.SemaphoreType.DMA,
        ],
    )(qkv, src_p)

    meta_q = jnp.stack([dst_p.reshape(nb, B), pos_p.reshape(nb, B)], axis=2)
    meta_k = jnp.stack([dst_p.reshape(nb, B), pos_p.reshape(nb, B)], axis=1)

    nb2 = nb // 2
    gm1 = lambda b: (jnp.maximum(2 * b - 1, 0), 0)
    g0m = lambda b: (2 * b, 0)
    g1m = lambda b: (2 * b + 1, 0)
    g2m = lambda b: (jnp.minimum(2 * b + 2, nb - 1), 0)
    vps, sps, mout = pl.pallas_call(
        functools.partial(_attn_body, nb2),
        grid=(nb2,),
        in_specs=[
            pl.BlockSpec((B, 3 * d), gm1),
            pl.BlockSpec((B, 3 * d), g0m),
            pl.BlockSpec((B, 3 * d), g1m),
            pl.BlockSpec((B, 3 * d), g2m),
            pl.BlockSpec((1, B, 2), lambda b: (2 * b, 0, 0)),
            pl.BlockSpec((1, B, 2), lambda b: (2 * b + 1, 0, 0)),
            pl.BlockSpec((1, 2, B), lambda b: (jnp.maximum(2 * b - 1, 0), 0, 0)),
            pl.BlockSpec((1, 2, B), lambda b: (2 * b, 0, 0)),
            pl.BlockSpec((1, 2, B), lambda b: (2 * b + 1, 0, 0)),
            pl.BlockSpec((1, 2, B), lambda b: (jnp.minimum(2 * b + 2, nb - 1), 0, 0)),
            pl.BlockSpec((d, d), lambda b: (0, 0)),
            pl.BlockSpec((d, d), lambda b: (0, 0)),
            pl.BlockSpec((d, d), lambda b: (0, 0)),
            pl.BlockSpec((8, d), lambda b: (0, 0)),
        ],
        out_specs=[pl.BlockSpec((2 * B, d), lambda b: (b, 0)),
                   pl.BlockSpec((2 * B, 1), lambda b: (b, 0)),
                   pl.BlockSpec((1, d), lambda b: (0, 0))],
        out_shape=[jax.ShapeDtypeStruct((ep, d), jnp.float32),
                   jax.ShapeDtypeStruct((ep, 1), jnp.float32),
                   jax.ShapeDtypeStruct((1, d), jnp.float32)],
        scratch_shapes=[pltpu.SMEM((1, 1), jnp.float32)],
    )(g_edge, g_edge, g_edge, g_edge, meta_q, meta_q, meta_k, meta_k, meta_k,
      meta_k, enc_W[3], pma_lin_W, pma_W[2], bias_pack)

    mvec = mout[0, :16]
    split = n // 2
    np2 = -(-max(split, n + 1 - split) // 128) * 128
    nacc = -(-(np2 + np2 // 8 + 128) // 2048) * 2048
    first_hi = jnp.searchsorted(dst_p, split).astype(jnp.int32)
    cb_vec = jnp.broadcast_to((first_hi + CH - 1) // CH, (16,))
    acc2 = pl.kernel(
        functools.partial(_segreduce_body, split, np2, nacc),
        mesh=mesh,
        out_type=jax.ShapeDtypeStruct((2, nacc, d), jnp.float32),
        scratch_types=[
            pltpu.VMEM((CH,), jnp.int32),
            pltpu.VMEM((CH,), jnp.int32),
            pltpu.VMEM((CH,), jnp.int32),
            pltpu.VMEM((CH,), jnp.float32),
            pltpu.VMEM((CH,), jnp.int32),
            pltpu.VMEM((CH,), jnp.float32),
            pltpu.VMEM((CH, d), jnp.float32),
            pltpu.VMEM((CH, d), jnp.float32),
            pltpu.VMEM((CH, d), jnp.float32),
            pltpu.VMEM((16,), jnp.float32),
            pltpu.VMEM((16,), jnp.int32),
            pltpu.VMEM_SHARED((nacc, d), jnp.float32),
        ],
    )(sps.reshape(ep), dst_p, pos_p, vps, mvec, cb_vec)

    den = acc2[:, np2:np2 + np2 // 8].reshape(2 * np2, 16)

    rb4 = 1024 if np2 % 1024 == 0 else 128
    nb2 = np2 // rb4
    out = pl.pallas_call(
        _tail_body,
        grid=(2 * nb2,),
        in_specs=[
            pl.BlockSpec((1, rb4, d), lambda i: (i // nb2, i % nb2, 0)),
            pl.BlockSpec((rb4, 16), lambda i: (i, 0)),
            pl.BlockSpec((1, d), lambda i: (0, 0)),
            pl.BlockSpec((d, d), lambda i: (0, 0)),
            pl.BlockSpec((d, d), lambda i: (0, 0)),
            pl.BlockSpec((d, d), lambda i: (0, 0)),
            pl.BlockSpec((8, d), lambda i: (0, 0)),
        ],
        out_specs=pl.BlockSpec((rb4, d), lambda i: (i, 0)),
        out_shape=jax.ShapeDtypeStruct((2 * np2, d), jnp.float32),
    )(acc2, den, qseed, pma_W[3], wd02, dec_W[3], tail_bias)
    return jnp.concatenate([out[:split], out[np2:np2 + n - split]])
